# baseline (speedup 1.0000x reference)
import functools
import math

import jax
import jax.numpy as jnp
from jax.experimental import pallas as pl

SEL = 64
N_ITER = 4
NEG = -3.0e38


def _sums_body(q_ref, k_ref, out_ref):
    ones = jnp.ones((64, 8), jnp.float32)
    ks = jax.lax.dot_general(k_ref[0, 0], ones, (((1,), (0,)), ((), ())),
                             preferred_element_type=jnp.float32)
    qs = jax.lax.dot_general(q_ref[0, 0], ones, (((1,), (0,)), ((), ())),
                             preferred_element_type=jnp.float32)
    out_ref[0, 0, :, :] = jnp.concatenate([ks, qs], axis=-1)


def _select_body(s_ref, idx_ref):
    R, N = s_ref.shape
    vals0 = s_ref[...]
    lane = jax.lax.broadcasted_iota(jnp.int32, (R, N), 1)
    col = jax.lax.broadcasted_iota(jnp.int32, (R, SEL), 1)

    def body(i, carry):
        vals, acc = carry
        idx = jnp.argmax(vals, axis=-1).astype(jnp.int32)
        acc = jnp.where(col == i, idx[:, None], acc)
        vals = jnp.where(lane == idx[:, None], NEG, vals)
        return vals, acc

    _, acc = jax.lax.fori_loop(
        0, SEL, body, (vals0, jnp.zeros((R, SEL), jnp.int32)))
    idx_ref[...] = acc


def _mm(a, b, ca, cb):
    return jax.lax.dot_general(
        a, b, ((( ca,), (cb,)), ((), ())),
        preferred_element_type=jnp.float32)


def _head(qs, k, v, idx_k, idx_q):
    N = qs.shape[0]
    lane = jax.lax.broadcasted_iota(jnp.int32, (SEL, N), 1)
    oh_k = (lane == idx_k[:, None]).astype(jnp.float32)
    oh_q = (lane == idx_q[:, None]).astype(jnp.float32)

    nc = _mm(oh_k, k, 1, 0)
    nr = _mm(oh_q, qs, 1, 0)

    E = jnp.exp(_mm(qs, nc, 1, 1))
    S = jnp.sum(E, axis=-1, keepdims=True)

    eu = jnp.exp(_mm(nr, nc, 1, 1))
    u = eu / jnp.sum(eu, axis=-1, keepdims=True)

    er = jnp.exp(_mm(nr, k, 1, 1))
    k3 = er / jnp.sum(er, axis=-1, keepdims=True)

    rv = _mm(k3, v, 1, 0)

    eye = (jax.lax.broadcasted_iota(jnp.int32, (SEL, SEL), 0)
           == jax.lax.broadcasted_iota(jnp.int32, (SEL, SEL), 1)
           ).astype(jnp.float32)
    denom = jnp.max(jnp.sum(u, axis=0))
    vinv = _mm(u, eye, 0, 0) * (1.0 / denom)
    for _ in range(N_ITER):
        kv = _mm(u, vinv, 1, 0)
        a1 = 7.0 * eye - kv
        a2 = 15.0 * eye - _mm(kv, a1, 1, 0)
        a3 = 13.0 * eye - _mm(kv, a2, 1, 0)
        vinv = 0.25 * _mm(vinv, a3, 1, 0)

    m = _mm(vinv, rv, 1, 0)
    return _mm(E, m, 1, 0) / S


def _main_body(q_ref, k_ref, v_ref, ik_ref, iq_ref, out_ref):
    scale = jnp.float32(1.0 / math.sqrt(64.0))
    for j in range(q_ref.shape[1]):
        out_ref[0, j] = _head(q_ref[0, j] * scale, k_ref[0, j], v_ref[0, j],
                              ik_ref[0, j], iq_ref[0, j])


HB = 2


def kernel(Q, K, V, mask):
    B, H, N, D = Q.shape
    BH = B * H

    sums = pl.pallas_call(
        _sums_body,
        grid=(BH,),
        in_specs=[
            pl.BlockSpec((1, 1, N, D), lambda i: (i // H, i % H, 0, 0)),
            pl.BlockSpec((1, 1, N, D), lambda i: (i // H, i % H, 0, 0)),
        ],
        out_specs=pl.BlockSpec((1, 1, N, 16), lambda i: (i // H, i % H, 0, 0)),
        out_shape=jax.ShapeDtypeStruct((B, H, N, 16), jnp.float32),
    )(Q, K)

    rows = sums[..., ::8]
    rows = jnp.moveaxis(rows, -1, -2).reshape(2 * BH, N)

    idx = pl.pallas_call(
        _select_body,
        out_shape=jax.ShapeDtypeStruct((2 * BH, SEL), jnp.int32),
    )(rows)

    idx = idx.reshape(B * H // HB, HB, 2, SEL)
    idx_k = idx[:, :, 0, :]
    idx_q = idx[:, :, 1, :]

    HBLK = H // HB
    X = pl.pallas_call(
        _main_body,
        grid=(BH // HB,),
        in_specs=[
            pl.BlockSpec((1, HB, N, D), lambda i: (i // HBLK, i % HBLK, 0, 0)),
            pl.BlockSpec((1, HB, N, D), lambda i: (i // HBLK, i % HBLK, 0, 0)),
            pl.BlockSpec((1, HB, N, D), lambda i: (i // HBLK, i % HBLK, 0, 0)),
            pl.BlockSpec((1, HB, SEL), lambda i: (i, 0, 0)),
            pl.BlockSpec((1, HB, SEL), lambda i: (i, 0, 0)),
        ],
        out_specs=pl.BlockSpec((1, HB, N, D), lambda i: (i // HBLK, i % HBLK, 0, 0)),
        out_shape=jax.ShapeDtypeStruct((B, H, N, D), jnp.float32),
    )(Q, K, V, idx_k, idx_q)

    return X


# speedup vs baseline: 1.6773x; 1.6773x over previous
import functools
import math

import jax
import jax.numpy as jnp
from jax.experimental import pallas as pl

SEL = 64
N_ITER = 4
NEG = -3.0e38


def _sums_body(q_ref, k_ref, out_ref):
    out_ref[0, 0, 0, :] = jnp.sum(k_ref[0, 0], axis=-1)
    out_ref[0, 0, 1, :] = jnp.sum(q_ref[0, 0], axis=-1)


def _select_body(s_ref, idx_ref):
    R, N = s_ref.shape
    vals0 = s_ref[...]
    lane = jax.lax.broadcasted_iota(jnp.int32, (R, N), 1)
    col = jax.lax.broadcasted_iota(jnp.int32, (R, SEL), 1)

    def body(i, carry):
        vals, acc = carry
        idx = jnp.argmax(vals, axis=-1).astype(jnp.int32)
        acc = jnp.where(col == i, idx[:, None], acc)
        vals = jnp.where(lane == idx[:, None], NEG, vals)
        return vals, acc

    _, acc = jax.lax.fori_loop(
        0, SEL, body, (vals0, jnp.zeros((R, SEL), jnp.int32)))
    idx_ref[...] = acc


def _mm(a, b, ca, cb):
    return jax.lax.dot_general(
        a, b, ((( ca,), (cb,)), ((), ())),
        preferred_element_type=jnp.float32)


def _head(qs, k, v, idx_k, idx_q):
    N = qs.shape[0]
    lane = jax.lax.broadcasted_iota(jnp.int32, (SEL, N), 1)
    oh_k = (lane == idx_k[:, None]).astype(jnp.float32)
    oh_q = (lane == idx_q[:, None]).astype(jnp.float32)

    nc = _mm(oh_k, k, 1, 0)
    nr = _mm(oh_q, qs, 1, 0)

    c = _mm(qs, nc, 1, 1)
    E = jnp.exp(c - jnp.max(c, axis=-1, keepdims=True))
    S = jnp.sum(E, axis=-1, keepdims=True)

    cu = _mm(nr, nc, 1, 1)
    eu = jnp.exp(cu - jnp.max(cu, axis=-1, keepdims=True))
    u = eu / jnp.sum(eu, axis=-1, keepdims=True)

    r = _mm(nr, k, 1, 1)
    er = jnp.exp(r - jnp.max(r, axis=-1, keepdims=True))
    k3 = er / jnp.sum(er, axis=-1, keepdims=True)

    rv = _mm(k3, v, 1, 0)

    eye = (jax.lax.broadcasted_iota(jnp.int32, (SEL, SEL), 0)
           == jax.lax.broadcasted_iota(jnp.int32, (SEL, SEL), 1)
           ).astype(jnp.float32)
    denom = jnp.max(jnp.sum(u, axis=0))
    vinv = _mm(u, eye, 0, 0) * (1.0 / denom)
    for _ in range(N_ITER):
        kv = _mm(u, vinv, 1, 0)
        a1 = 7.0 * eye - kv
        a2 = 15.0 * eye - _mm(kv, a1, 1, 0)
        a3 = 13.0 * eye - _mm(kv, a2, 1, 0)
        vinv = 0.25 * _mm(vinv, a3, 1, 0)

    m = _mm(vinv, rv, 1, 0)
    return _mm(E, m, 1, 0) / S


def _main_body(q_ref, k_ref, v_ref, ik_ref, iq_ref, out_ref):
    scale = jnp.float32(1.0 / math.sqrt(64.0))
    for j in range(q_ref.shape[1]):
        out_ref[0, j] = _head(q_ref[0, j] * scale, k_ref[0, j], v_ref[0, j],
                              ik_ref[0, j], iq_ref[0, j])


HB = 2


def kernel(Q, K, V, mask):
    B, H, N, D = Q.shape
    BH = B * H

    sums = pl.pallas_call(
        _sums_body,
        grid=(BH,),
        in_specs=[
            pl.BlockSpec((1, 1, N, D), lambda i: (i // H, i % H, 0, 0)),
            pl.BlockSpec((1, 1, N, D), lambda i: (i // H, i % H, 0, 0)),
        ],
        out_specs=pl.BlockSpec((1, 1, 2, N), lambda i: (i // H, i % H, 0, 0)),
        out_shape=jax.ShapeDtypeStruct((B, H, 2, N), jnp.float32),
    )(Q, K)

    rows = sums.reshape(2 * BH, N)

    idx = pl.pallas_call(
        _select_body,
        out_shape=jax.ShapeDtypeStruct((2 * BH, SEL), jnp.int32),
    )(rows)

    idx = idx.reshape(B * H // HB, HB, 2, SEL)
    idx_k = idx[:, :, 0, :]
    idx_q = idx[:, :, 1, :]

    HBLK = H // HB
    X = pl.pallas_call(
        _main_body,
        grid=(BH // HB,),
        in_specs=[
            pl.BlockSpec((1, HB, N, D), lambda i: (i // HBLK, i % HBLK, 0, 0)),
            pl.BlockSpec((1, HB, N, D), lambda i: (i // HBLK, i % HBLK, 0, 0)),
            pl.BlockSpec((1, HB, N, D), lambda i: (i // HBLK, i % HBLK, 0, 0)),
            pl.BlockSpec((1, HB, SEL), lambda i: (i, 0, 0)),
            pl.BlockSpec((1, HB, SEL), lambda i: (i, 0, 0)),
        ],
        out_specs=pl.BlockSpec((1, HB, N, D), lambda i: (i // HBLK, i % HBLK, 0, 0)),
        out_shape=jax.ShapeDtypeStruct((B, H, N, D), jnp.float32),
    )(Q, K, V, idx_k, idx_q)

    return X
